# baseline (speedup 1.0000x reference)
import functools

import jax
import jax.numpy as jnp
from jax import lax
from jax.experimental import pallas as pl
from jax.experimental.pallas import tpu as pltpu
from jax.experimental.pallas import tpu_sc as plsc

N_V = 400002
N_D = 200
D_PAD = 256
TAIL = N_D - 128

NC = 2
NS = 16
NW = NC * NS

CHUNK = 128

BLK_V = 8192


def _transpose_body(wt_ref, eye_ref, out_ref):
    x = wt_ref[...]
    r = lax.dot_general(x, eye_ref[...], (((0,), (0,)), ((), ())),
                        preferred_element_type=jnp.float32)
    out_ref[...] = r


def _transpose_tc(wt, eye, half):
    grid = -(-N_V // BLK_V)
    return pl.pallas_call(
        _transpose_body,
        grid=(grid,),
        in_specs=[
            pl.BlockSpec((128, BLK_V), lambda j, h=half: (h, j)),
            pl.BlockSpec((128, 128), lambda j: (0, 0)),
        ],
        out_specs=pl.BlockSpec((BLK_V, 128), lambda j: (j, 0)),
        out_shape=jax.ShapeDtypeStruct((N_V, 128), jnp.float32),
    )(wt, eye)


def _gather_body(idx_hbm, ta_hbm, tb_hbm, out_hbm,
                 idx_v, buf_a, buf_b, buf_t,
                 sem_a0, sem_a1, sem_b0, sem_b1,
                 sem_oa0, sem_oa1, sem_ot0, sem_ot1):
    c = lax.axis_index("c")
    s = lax.axis_index("s")
    wid = s * NC + c
    n_chunks = idx_v.shape[0]
    pltpu.sync_copy(idx_hbm.at[wid], idx_v)
    base = wid * (n_chunks * CHUNK)
    sems_a = (sem_a0, sem_a1)
    sems_b = (sem_b0, sem_b1)
    sems_oa = (sem_oa0, sem_oa1)
    sems_ot = (sem_ot0, sem_ot1)

    def start(j, slot):
        pltpu.async_copy(ta_hbm.at[idx_v.at[j]], buf_a.at[slot], sems_a[slot])
        pltpu.async_copy(tb_hbm.at[idx_v.at[j]], buf_b.at[slot], sems_b[slot])

    def wait(slot):
        pltpu.make_async_copy(
            ta_hbm.at[idx_v.at[0]], buf_a.at[slot], sems_a[slot]).wait()
        pltpu.make_async_copy(
            tb_hbm.at[idx_v.at[0]], buf_b.at[slot], sems_b[slot]).wait()

    def out_slices(j):
        rows = pl.ds(base + j * CHUNK, CHUNK)
        return out_hbm.at[rows, pl.ds(0, 128)], out_hbm.at[rows, pl.ds(128, TAIL)]

    def wait_oa(slot):
        dst_a, _ = out_slices(0)
        pltpu.make_async_copy(buf_a.at[slot], dst_a, sems_oa[slot]).wait()

    def wait_ot(slot):
        _, dst_t = out_slices(0)
        pltpu.make_async_copy(buf_t.at[slot], dst_t, sems_ot[slot]).wait()

    def repack(slot):
        bb = buf_b.at[slot]
        bt = buf_t.at[slot]

        def row(i2, carry):
            for r in range(2):
                i = i2 * 2 + r
                for k in range(4):
                    bt[i, pl.ds(16 * k, 16)] = bb[i, pl.ds(16 * k, 16)]
                bt[i, pl.ds(TAIL - 16, 16)] = bb[i, pl.ds(TAIL - 16, 16)]
            return carry

        lax.fori_loop(0, CHUNK // 2, row, 0)

    start(0, 0)

    def group(g, carry):
        for b in range(2):
            j = 2 * g + b
            wait(b)

            @pl.when(j + 1 < n_chunks)
            def _():
                @pl.when(j >= 1)
                def _():
                    wait_oa(1 - b)

                start(j + 1, 1 - b)

            dst_a, dst_t = out_slices(j)
            pltpu.async_copy(buf_a.at[b], dst_a, sems_oa[b])

            @pl.when(j >= 2)
            def _():
                wait_ot(b)

            repack(b)
            pltpu.async_copy(buf_t.at[b], dst_t, sems_ot[b])
        return carry

    lax.fori_loop(0, n_chunks // 2, group, 0)

    def drain(g, carry):
        for b in range(2):
            wait_oa(b)
            wait_ot(b)
        return carry

    lax.fori_loop(0, 1, drain, 0)


@functools.partial(jax.jit, static_argnames=("n_chunks",))
def _run(idx3, wt, eye, n_chunks):
    t2a = _transpose_tc(wt, eye, 0)
    t2b = _transpose_tc(wt, eye, 1)
    mesh = plsc.VectorSubcoreMesh(
        core_axis_name="c", subcore_axis_name="s", num_cores=NC, num_subcores=NS
    )
    total = NW * n_chunks * CHUNK
    return pl.kernel(
        _gather_body,
        out_type=jax.ShapeDtypeStruct((total, N_D), jnp.float32),
        mesh=mesh,
        scratch_types=[
            pltpu.VMEM((n_chunks, CHUNK), jnp.int32),
            pltpu.VMEM((2, CHUNK, 128), jnp.float32),
            pltpu.VMEM((2, CHUNK, 128), jnp.float32),
            pltpu.VMEM((2, CHUNK, TAIL), jnp.float32),
            pltpu.SemaphoreType.DMA,
            pltpu.SemaphoreType.DMA,
            pltpu.SemaphoreType.DMA,
            pltpu.SemaphoreType.DMA,
            pltpu.SemaphoreType.DMA,
            pltpu.SemaphoreType.DMA,
            pltpu.SemaphoreType.DMA,
            pltpu.SemaphoreType.DMA,
        ],
    )(idx3, t2a, t2b)


def kernel(input, weight):
    B, S = input.shape
    total = B * S
    assert total % (NW * CHUNK) == 0
    n_chunks = total // (NW * CHUNK)
    assert n_chunks % 2 == 0
    idx3 = input.reshape(NW, n_chunks, CHUNK).astype(jnp.int32)
    wt = weight.T
    eye = jnp.eye(128, dtype=jnp.float32)
    out = _run(idx3, wt, eye, n_chunks)
    return out.reshape(B, S, N_D)


# speedup vs baseline: 1.0114x; 1.0114x over previous
import functools

import jax
import jax.numpy as jnp
from jax import lax
from jax.experimental import pallas as pl
from jax.experimental.pallas import tpu as pltpu
from jax.experimental.pallas import tpu_sc as plsc

N_V = 400002
N_D = 200
D_PAD = 256
TAIL = N_D - 128

NC = 2
NS = 16
NW = NC * NS

CHUNK = 128

BLK_V = 16384


def _transpose_body(wt_ref, eye_ref, out_ref):
    x = wt_ref[...]
    r = lax.dot_general(x, eye_ref[...], (((0,), (0,)), ((), ())),
                        preferred_element_type=jnp.float32)
    out_ref[...] = r


def _transpose_tc(wt, eye, half):
    grid = -(-N_V // BLK_V)
    return pl.pallas_call(
        _transpose_body,
        grid=(grid,),
        in_specs=[
            pl.BlockSpec((128, BLK_V), lambda j, h=half: (h, j)),
            pl.BlockSpec((128, 128), lambda j: (0, 0)),
        ],
        out_specs=pl.BlockSpec((BLK_V, 128), lambda j: (j, 0)),
        out_shape=jax.ShapeDtypeStruct((N_V, 128), jnp.float32),
    )(wt, eye)


def _gather_body(idx_hbm, ta_hbm, tb_hbm, out_hbm,
                 idx_v, buf_a, buf_b, buf_t,
                 sem_a0, sem_a1, sem_b0, sem_b1,
                 sem_oa0, sem_oa1, sem_ot0, sem_ot1):
    c = lax.axis_index("c")
    s = lax.axis_index("s")
    wid = s * NC + c
    n_chunks = idx_v.shape[0]
    pltpu.sync_copy(idx_hbm.at[wid], idx_v)
    base = wid * (n_chunks * CHUNK)
    sems_a = (sem_a0, sem_a1)
    sems_b = (sem_b0, sem_b1)
    sems_oa = (sem_oa0, sem_oa1)
    sems_ot = (sem_ot0, sem_ot1)

    def start(j, slot):
        pltpu.async_copy(ta_hbm.at[idx_v.at[j]], buf_a.at[slot], sems_a[slot])
        pltpu.async_copy(tb_hbm.at[idx_v.at[j]], buf_b.at[slot], sems_b[slot])

    def wait(slot):
        pltpu.make_async_copy(
            ta_hbm.at[idx_v.at[0]], buf_a.at[slot], sems_a[slot]).wait()
        pltpu.make_async_copy(
            tb_hbm.at[idx_v.at[0]], buf_b.at[slot], sems_b[slot]).wait()

    def out_slices(j):
        rows = pl.ds(base + j * CHUNK, CHUNK)
        return out_hbm.at[rows, pl.ds(0, 128)], out_hbm.at[rows, pl.ds(128, TAIL)]

    def wait_oa(slot):
        dst_a, _ = out_slices(0)
        pltpu.make_async_copy(buf_a.at[slot], dst_a, sems_oa[slot]).wait()

    def wait_ot(slot):
        _, dst_t = out_slices(0)
        pltpu.make_async_copy(buf_t.at[slot], dst_t, sems_ot[slot]).wait()

    def repack(slot):
        bb = buf_b.at[slot]
        bt = buf_t.at[slot]

        def row(i2, carry):
            for r in range(2):
                i = i2 * 2 + r
                for k in range(4):
                    bt[i, pl.ds(16 * k, 16)] = bb[i, pl.ds(16 * k, 16)]
                bt[i, pl.ds(TAIL - 16, 16)] = bb[i, pl.ds(TAIL - 16, 16)]
            return carry

        lax.fori_loop(0, CHUNK // 2, row, 0)

    start(0, 0)

    def group(g, carry):
        for b in range(2):
            j = 2 * g + b
            wait(b)

            @pl.when(j + 1 < n_chunks)
            def _():
                @pl.when(j >= 1)
                def _():
                    wait_oa(1 - b)

                start(j + 1, 1 - b)

            dst_a, dst_t = out_slices(j)
            pltpu.async_copy(buf_a.at[b], dst_a, sems_oa[b])

            @pl.when(j >= 2)
            def _():
                wait_ot(b)

            repack(b)
            pltpu.async_copy(buf_t.at[b], dst_t, sems_ot[b])
        return carry

    lax.fori_loop(0, n_chunks // 2, group, 0)

    def drain(g, carry):
        for b in range(2):
            wait_oa(b)
            wait_ot(b)
        return carry

    lax.fori_loop(0, 1, drain, 0)


@functools.partial(jax.jit, static_argnames=("n_chunks",))
def _run(idx3, wt, eye, n_chunks):
    t2a = _transpose_tc(wt, eye, 0)
    t2b = _transpose_tc(wt, eye, 1)
    mesh = plsc.VectorSubcoreMesh(
        core_axis_name="c", subcore_axis_name="s", num_cores=NC, num_subcores=NS
    )
    total = NW * n_chunks * CHUNK
    return pl.kernel(
        _gather_body,
        out_type=jax.ShapeDtypeStruct((total, N_D), jnp.float32),
        mesh=mesh,
        scratch_types=[
            pltpu.VMEM((n_chunks, CHUNK), jnp.int32),
            pltpu.VMEM((2, CHUNK, 128), jnp.float32),
            pltpu.VMEM((2, CHUNK, 128), jnp.float32),
            pltpu.VMEM((2, CHUNK, TAIL), jnp.float32),
            pltpu.SemaphoreType.DMA,
            pltpu.SemaphoreType.DMA,
            pltpu.SemaphoreType.DMA,
            pltpu.SemaphoreType.DMA,
            pltpu.SemaphoreType.DMA,
            pltpu.SemaphoreType.DMA,
            pltpu.SemaphoreType.DMA,
            pltpu.SemaphoreType.DMA,
        ],
    )(idx3, t2a, t2b)


def kernel(input, weight):
    B, S = input.shape
    total = B * S
    assert total % (NW * CHUNK) == 0
    n_chunks = total // (NW * CHUNK)
    assert n_chunks % 2 == 0
    idx3 = input.reshape(NW, n_chunks, CHUNK).astype(jnp.int32)
    wt = weight.T
    eye = jnp.eye(128, dtype=jnp.float32)
    out = _run(idx3, wt, eye, n_chunks)
    return out.reshape(B, S, N_D)
